# baseline (speedup 1.0000x reference)
import functools

import jax
import jax.numpy as jnp
from jax import lax
from jax.experimental import pallas as pl
from jax.experimental.pallas import tpu as pltpu
from jax.experimental.pallas import tpu_sc as plsc

N_NODES = 10000
D_FEAT = 128
N_PAIR = D_FEAT // 2
N_EDGES = 320000

NUM_CORES = 2
NUM_SUBCORES = 16
NUM_WORKERS = NUM_CORES * NUM_SUBCORES
EDGES_PER_WORKER = N_EDGES // NUM_WORKERS
CHUNK = 80
NCHUNK = EDGES_PER_WORKER // CHUNK
GROUPS = CHUNK // 16
STAGE_ROWS = 200
N_STAGE_PIECES = N_NODES // STAGE_ROWS

_mesh = plsc.VectorSubcoreMesh(core_axis_name="c", subcore_axis_name="s")


@functools.partial(
    pl.kernel,
    mesh=_mesh,
    compiler_params=pltpu.CompilerParams(
        needs_layout_passes=False, use_tc_tiling_on_sc=False),
    out_type=jax.ShapeDtypeStruct((N_EDGES,), jnp.float32),
    scratch_types=[
        pltpu.VMEM((NCHUNK, CHUNK), jnp.int32),
        pltpu.VMEM((NCHUNK, CHUNK), jnp.int32),
        pltpu.VMEM((CHUNK, N_PAIR), jnp.float32),
        pltpu.VMEM((CHUNK, N_PAIR), jnp.float32),
        pltpu.VMEM((CHUNK, N_PAIR), jnp.float32),
        pltpu.VMEM((CHUNK, N_PAIR), jnp.float32),
        pltpu.VMEM((N_PAIR, 32), jnp.bfloat16),
        pltpu.VMEM((EDGES_PER_WORKER,), jnp.float32),
        pltpu.VMEM((STAGE_ROWS, N_PAIR), jnp.float32),
        pltpu.VMEM_SHARED((N_NODES, N_PAIR), jnp.float32),
        pltpu.SemaphoreType.DMA((2,)),
        pltpu.SemaphoreType.DMA((2,)),
    ],
)
def _decoder_sc(nf_hbm, r_hbm, c_hbm, wb_hbm, out_hbm,
                ri_v, ci_v, rr0, rr1, cr0, cr1, wb_v, out_v, stage_v,
                table_sh, sem_r, sem_c):
    sid = lax.axis_index("s")
    wid = sid * NUM_CORES + lax.axis_index("c")
    pltpu.sync_copy(wb_hbm, wb_v)
    pltpu.sync_copy(r_hbm.at[wid], ri_v)
    pltpu.sync_copy(c_hbm.at[wid], ci_v)

    def stage_body(k, carry):
        piece = sid + k * NUM_SUBCORES

        @pl.when(piece < N_STAGE_PIECES)
        def _():
            off = piece * STAGE_ROWS
            pltpu.sync_copy(nf_hbm.at[pl.ds(off, STAGE_ROWS)], stage_v)
            pltpu.sync_copy(stage_v, table_sh.at[pl.ds(off, STAGE_ROWS)])

        return carry

    lax.fori_loop(0, (N_STAGE_PIECES + NUM_SUBCORES - 1) // NUM_SUBCORES,
                  stage_body, 0)
    plsc.subcore_barrier()

    base = wid * EDGES_PER_WORKER
    lane = lax.iota(jnp.int32, 16)
    bufs = ((rr0, cr0), (rr1, cr1))

    def start_gathers(i, b):
        rr, cr = bufs[b]
        pltpu.async_copy(table_sh.at[ri_v.at[i]], rr, sem_r.at[b])
        pltpu.async_copy(table_sh.at[ci_v.at[i]], cr, sem_c.at[b])

    def wait_gathers(i, b):
        rr, cr = bufs[b]
        pltpu.make_async_copy(table_sh.at[ri_v.at[i]], rr,
                              sem_r.at[b]).wait()
        pltpu.make_async_copy(table_sh.at[ci_v.at[i]], cr,
                              sem_c.at[b]).wait()

    def compute_chunk(i, b):
        rr, cr = bufs[b]
        evecs = [lane + g * 16 for g in range(GROUPS)]

        def p_body(p, accs):
            wv = wb_v[p]
            pvec = (jnp.full((16,), p, dtype=jnp.int32) + lane) & (N_PAIR - 1)
            new = []
            for g in range(GROUPS):
                ar = plsc.load_gather(rr, [evecs[g], pvec])
                ac = plsc.load_gather(cr, [evecs[g], pvec])
                arb = plsc.bitcast(ar, jnp.bfloat16)
                acb = plsc.bitcast(ac, jnp.bfloat16)
                m = jnp.abs(arb - acb) * wv
                lo, hi = plsc.unpack(m, format=plsc.PackFormat.INTERLEAVED)
                new.append(accs[2 * g] + lo)
                new.append(accs[2 * g + 1] + hi)
            return tuple(new)

        zero = jnp.zeros((16,), jnp.float32)
        accs = lax.fori_loop(0, N_PAIR, p_body, (zero,) * (2 * GROUPS),
                             unroll=8)
        for g in range(GROUPS):
            plsc.store_scatter(out_v, [evecs[g] + i * CHUNK],
                               accs[2 * g] + accs[2 * g + 1])

    start_gathers(0, 0)

    def pair_body(j, carry):
        i0 = 2 * j
        start_gathers(i0 + 1, 1)
        wait_gathers(i0, 0)
        compute_chunk(i0, 0)
        start_gathers(i0 + 2, 0)
        wait_gathers(i0 + 1, 1)
        compute_chunk(i0 + 1, 1)
        return carry

    lax.fori_loop(0, (NCHUNK - 1) // 2, pair_body, 0)
    wait_gathers(NCHUNK - 1, 0)
    compute_chunk(NCHUNK - 1, 0)
    pltpu.sync_copy(out_v, out_hbm.at[pl.ds(base, EDGES_PER_WORKER)])


def kernel(node_features, r_indices, c_indices, w):
    r = r_indices.astype(jnp.int32).reshape(NUM_WORKERS, NCHUNK, CHUNK)
    c = c_indices.astype(jnp.int32).reshape(NUM_WORKERS, NCHUNK, CHUNK)
    nf_bf = node_features.astype(jnp.bfloat16)
    nf_packed = lax.bitcast_convert_type(
        nf_bf.reshape(N_NODES, N_PAIR, 2), jnp.float32)
    wf = w.reshape(D_FEAT).astype(jnp.bfloat16)
    q = (jnp.arange(N_PAIR)[:, None] + jnp.arange(16)[None, :]) % N_PAIR
    wb = jnp.stack([wf[2 * q], wf[2 * q + 1]], axis=-1).reshape(N_PAIR, 32)
    return _decoder_sc(nf_packed, r, c, wb)


# speedup vs baseline: 1.3161x; 1.3161x over previous
import functools

import jax
import jax.numpy as jnp
from jax import lax
from jax.experimental import pallas as pl
from jax.experimental.pallas import tpu as pltpu
from jax.experimental.pallas import tpu_sc as plsc

N_NODES = 10000
D_FEAT = 128
N_PAIR = D_FEAT // 2
N_EDGES = 320000

NUM_CORES = 2
NUM_SUBCORES = 16
NUM_WORKERS = NUM_CORES * NUM_SUBCORES
EDGES_PER_WORKER = N_EDGES // NUM_WORKERS
CHUNK = 80
NCHUNK = EDGES_PER_WORKER // CHUNK
GROUPS = CHUNK // 16
STAGE_ROWS = 200
N_STAGE_PIECES = N_NODES // STAGE_ROWS

_mesh = plsc.VectorSubcoreMesh(core_axis_name="c", subcore_axis_name="s")


@functools.partial(
    pl.kernel,
    mesh=_mesh,
    compiler_params=pltpu.CompilerParams(
        needs_layout_passes=False, use_tc_tiling_on_sc=False),
    out_type=jax.ShapeDtypeStruct((N_EDGES,), jnp.float32),
    scratch_types=[
        pltpu.VMEM((NCHUNK, CHUNK), jnp.int32),
        pltpu.VMEM((NCHUNK, CHUNK), jnp.int32),
        pltpu.VMEM((CHUNK, N_PAIR), jnp.float32),
        pltpu.VMEM((CHUNK, N_PAIR), jnp.float32),
        pltpu.VMEM((CHUNK, N_PAIR), jnp.float32),
        pltpu.VMEM((CHUNK, N_PAIR), jnp.float32),
        pltpu.VMEM((N_PAIR, 32), jnp.bfloat16),
        pltpu.VMEM((EDGES_PER_WORKER,), jnp.float32),
        pltpu.VMEM((STAGE_ROWS, N_PAIR), jnp.float32),
        pltpu.VMEM_SHARED((N_NODES, N_PAIR), jnp.float32),
        pltpu.SemaphoreType.DMA((2,)),
        pltpu.SemaphoreType.DMA((2,)),
    ],
)
def _decoder_sc(nf_hbm, r_hbm, c_hbm, wb_hbm, out_hbm,
                ri_v, ci_v, rr0, rr1, cr0, cr1, wb_v, out_v, stage_v,
                table_sh, sem_r, sem_c):
    sid = lax.axis_index("s")
    wid = sid * NUM_CORES + lax.axis_index("c")
    pltpu.sync_copy(wb_hbm, wb_v)
    pltpu.sync_copy(r_hbm.at[wid], ri_v)
    pltpu.sync_copy(c_hbm.at[wid], ci_v)

    def stage_body(k, carry):
        piece = sid + k * NUM_SUBCORES

        @pl.when(piece < N_STAGE_PIECES)
        def _():
            off = piece * STAGE_ROWS
            pltpu.sync_copy(nf_hbm.at[pl.ds(off, STAGE_ROWS)], stage_v)
            pltpu.sync_copy(stage_v, table_sh.at[pl.ds(off, STAGE_ROWS)])

        return carry

    lax.fori_loop(0, (N_STAGE_PIECES + NUM_SUBCORES - 1) // NUM_SUBCORES,
                  stage_body, 0)
    plsc.subcore_barrier()

    base = wid * EDGES_PER_WORKER
    lane = lax.iota(jnp.int32, 16)
    bufs = ((rr0, cr0), (rr1, cr1))

    def start_gathers(i, b):
        rr, cr = bufs[b]
        pltpu.async_copy(table_sh.at[ri_v.at[i]], rr, sem_r.at[b])
        pltpu.async_copy(table_sh.at[ci_v.at[i]], cr, sem_c.at[b])

    def wait_gathers(i, b):
        rr, cr = bufs[b]
        pltpu.make_async_copy(table_sh.at[ri_v.at[i]], rr,
                              sem_r.at[b]).wait()
        pltpu.make_async_copy(table_sh.at[ci_v.at[i]], cr,
                              sem_c.at[b]).wait()

    def compute_chunk(i, b):
        rr, cr = bufs[b]
        evecs = [lane + g * 16 for g in range(GROUPS)]

        def p_body(p, accs):
            wv = wb_v[p]
            pvec = (jnp.full((16,), p, dtype=jnp.int32) + lane) & (N_PAIR - 1)
            new = []
            for g in range(GROUPS):
                ar = plsc.load_gather(rr, [evecs[g], pvec])
                ac = plsc.load_gather(cr, [evecs[g], pvec])
                arb = plsc.bitcast(ar, jnp.bfloat16)
                acb = plsc.bitcast(ac, jnp.bfloat16)
                m = jnp.abs(arb - acb) * wv
                lo, hi = plsc.unpack(m, format=plsc.PackFormat.INTERLEAVED)
                new.append(accs[2 * g] + lo)
                new.append(accs[2 * g + 1] + hi)
            return tuple(new)

        zero = jnp.zeros((16,), jnp.float32)
        accs = lax.fori_loop(0, N_PAIR, p_body, (zero,) * (2 * GROUPS),
                             unroll=4)
        for g in range(GROUPS):
            plsc.store_scatter(out_v, [evecs[g] + i * CHUNK],
                               accs[2 * g] + accs[2 * g + 1])

    start_gathers(0, 0)

    def pair_body(j, carry):
        i0 = 2 * j
        start_gathers(i0 + 1, 1)
        wait_gathers(i0, 0)
        compute_chunk(i0, 0)
        start_gathers(i0 + 2, 0)
        wait_gathers(i0 + 1, 1)
        compute_chunk(i0 + 1, 1)
        return carry

    lax.fori_loop(0, (NCHUNK - 1) // 2, pair_body, 0)
    wait_gathers(NCHUNK - 1, 0)
    compute_chunk(NCHUNK - 1, 0)
    pltpu.sync_copy(out_v, out_hbm.at[pl.ds(base, EDGES_PER_WORKER)])


def kernel(node_features, r_indices, c_indices, w):
    r = r_indices.astype(jnp.int32).reshape(NUM_WORKERS, NCHUNK, CHUNK)
    c = c_indices.astype(jnp.int32).reshape(NUM_WORKERS, NCHUNK, CHUNK)
    nf_bf = node_features.astype(jnp.bfloat16)
    nf_packed = lax.bitcast_convert_type(
        nf_bf.reshape(N_NODES, N_PAIR, 2), jnp.float32)
    wf = w.reshape(D_FEAT).astype(jnp.bfloat16)
    q = (jnp.arange(N_PAIR)[:, None] + jnp.arange(16)[None, :]) % N_PAIR
    wb = jnp.stack([wf[2 * q], wf[2 * q + 1]], axis=-1).reshape(N_PAIR, 32)
    return _decoder_sc(nf_packed, r, c, wb)


# speedup vs baseline: 1.3483x; 1.0244x over previous
import functools

import jax
import jax.numpy as jnp
from jax import lax
from jax.experimental import pallas as pl
from jax.experimental.pallas import tpu as pltpu
from jax.experimental.pallas import tpu_sc as plsc

N_NODES = 10000
D_FEAT = 128
N_PAIR = D_FEAT // 2
N_EDGES = 320000

NUM_CORES = 2
NUM_SUBCORES = 16
NUM_WORKERS = NUM_CORES * NUM_SUBCORES
EDGES_PER_WORKER = N_EDGES // NUM_WORKERS
CHUNK = 80
NCHUNK = EDGES_PER_WORKER // CHUNK
GROUPS = CHUNK // 16
STAGE_ROWS = 200
N_STAGE_PIECES = N_NODES // STAGE_ROWS

_mesh = plsc.VectorSubcoreMesh(core_axis_name="c", subcore_axis_name="s")


@functools.partial(
    pl.kernel,
    mesh=_mesh,
    compiler_params=pltpu.CompilerParams(
        needs_layout_passes=False, use_tc_tiling_on_sc=False),
    out_type=jax.ShapeDtypeStruct((N_EDGES,), jnp.float32),
    scratch_types=[
        pltpu.VMEM((NCHUNK, CHUNK), jnp.int32),
        pltpu.VMEM((NCHUNK, CHUNK), jnp.int32),
        pltpu.VMEM((CHUNK, N_PAIR), jnp.float32),
        pltpu.VMEM((CHUNK, N_PAIR), jnp.float32),
        pltpu.VMEM((CHUNK, N_PAIR), jnp.float32),
        pltpu.VMEM((CHUNK, N_PAIR), jnp.float32),
        pltpu.VMEM((N_PAIR, 32), jnp.bfloat16),
        pltpu.VMEM((EDGES_PER_WORKER,), jnp.float32),
        pltpu.VMEM((STAGE_ROWS, N_PAIR), jnp.float32),
        pltpu.VMEM_SHARED((N_NODES, N_PAIR), jnp.float32),
        pltpu.SemaphoreType.DMA((2,)),
        pltpu.SemaphoreType.DMA((2,)),
    ],
)
def _decoder_sc(nf_hbm, r_hbm, c_hbm, wb_hbm, out_hbm,
                ri_v, ci_v, rr0, rr1, cr0, cr1, wb_v, out_v, stage_v,
                table_sh, sem_r, sem_c):
    sid = lax.axis_index("s")
    wid = sid * NUM_CORES + lax.axis_index("c")
    pltpu.sync_copy(wb_hbm, wb_v)
    pltpu.sync_copy(r_hbm.at[wid], ri_v)
    pltpu.sync_copy(c_hbm.at[wid], ci_v)

    def stage_body(k, carry):
        piece = sid + k * NUM_SUBCORES

        @pl.when(piece < N_STAGE_PIECES)
        def _():
            off = piece * STAGE_ROWS
            pltpu.sync_copy(nf_hbm.at[pl.ds(off, STAGE_ROWS)], stage_v)
            pltpu.sync_copy(stage_v, table_sh.at[pl.ds(off, STAGE_ROWS)])

        return carry

    lax.fori_loop(0, (N_STAGE_PIECES + NUM_SUBCORES - 1) // NUM_SUBCORES,
                  stage_body, 0)
    plsc.subcore_barrier()

    base = wid * EDGES_PER_WORKER
    lane = lax.iota(jnp.int32, 16)
    bufs = ((rr0, cr0), (rr1, cr1))

    def start_gathers(i, b):
        rr, cr = bufs[b]
        pltpu.async_copy(table_sh.at[ri_v.at[i]], rr, sem_r.at[b])
        pltpu.async_copy(table_sh.at[ci_v.at[i]], cr, sem_c.at[b])

    def wait_gathers(i, b):
        rr, cr = bufs[b]
        pltpu.make_async_copy(table_sh.at[ri_v.at[i]], rr,
                              sem_r.at[b]).wait()
        pltpu.make_async_copy(table_sh.at[ci_v.at[i]], cr,
                              sem_c.at[b]).wait()

    def compute_chunk(i, b):
        rr, cr = bufs[b]
        evecs = [lane + g * 16 for g in range(GROUPS)]

        def p_body(p, accs):
            wv = wb_v[p]
            pvec = (jnp.full((16,), p, dtype=jnp.int32) + lane) & (N_PAIR - 1)
            new = []
            for g in range(GROUPS):
                ar = plsc.load_gather(rr, [evecs[g], pvec])
                ac = plsc.load_gather(cr, [evecs[g], pvec])
                arb = plsc.bitcast(ar, jnp.bfloat16)
                acb = plsc.bitcast(ac, jnp.bfloat16)
                m = jnp.abs(arb - acb) * wv
                lo, hi = plsc.unpack(m, format=plsc.PackFormat.INTERLEAVED)
                new.append(accs[2 * g] + lo)
                new.append(accs[2 * g + 1] + hi)
            return tuple(new)

        zero = jnp.zeros((16,), jnp.float32)
        accs = lax.fori_loop(0, N_PAIR, p_body, (zero,) * (2 * GROUPS),
                             unroll=2)
        for g in range(GROUPS):
            plsc.store_scatter(out_v, [evecs[g] + i * CHUNK],
                               accs[2 * g] + accs[2 * g + 1])

    start_gathers(0, 0)

    def pair_body(j, carry):
        i0 = 2 * j
        start_gathers(i0 + 1, 1)
        wait_gathers(i0, 0)
        compute_chunk(i0, 0)
        start_gathers(i0 + 2, 0)
        wait_gathers(i0 + 1, 1)
        compute_chunk(i0 + 1, 1)
        return carry

    lax.fori_loop(0, (NCHUNK - 1) // 2, pair_body, 0)
    wait_gathers(NCHUNK - 1, 0)
    compute_chunk(NCHUNK - 1, 0)
    pltpu.sync_copy(out_v, out_hbm.at[pl.ds(base, EDGES_PER_WORKER)])


def kernel(node_features, r_indices, c_indices, w):
    r = r_indices.astype(jnp.int32).reshape(NUM_WORKERS, NCHUNK, CHUNK)
    c = c_indices.astype(jnp.int32).reshape(NUM_WORKERS, NCHUNK, CHUNK)
    nf_bf = node_features.astype(jnp.bfloat16)
    nf_packed = lax.bitcast_convert_type(
        nf_bf.reshape(N_NODES, N_PAIR, 2), jnp.float32)
    wf = w.reshape(D_FEAT).astype(jnp.bfloat16)
    q = (jnp.arange(N_PAIR)[:, None] + jnp.arange(16)[None, :]) % N_PAIR
    wb = jnp.stack([wf[2 * q], wf[2 * q + 1]], axis=-1).reshape(N_PAIR, 32)
    return _decoder_sc(nf_packed, r, c, wb)


# speedup vs baseline: 1.4051x; 1.0421x over previous
import functools

import jax
import jax.numpy as jnp
from jax import lax
from jax.experimental import pallas as pl
from jax.experimental.pallas import tpu as pltpu
from jax.experimental.pallas import tpu_sc as plsc

N_NODES = 10000
D_FEAT = 128
N_PAIR = D_FEAT // 2
N_EDGES = 320000

NUM_CORES = 2
NUM_SUBCORES = 16
NUM_WORKERS = NUM_CORES * NUM_SUBCORES
EDGES_PER_WORKER = N_EDGES // NUM_WORKERS
CHUNK = 80
NCHUNK = EDGES_PER_WORKER // CHUNK
GROUPS = CHUNK // 16
STAGE_ROWS = 200
N_STAGE_PIECES = N_NODES // STAGE_ROWS

_mesh = plsc.VectorSubcoreMesh(core_axis_name="c", subcore_axis_name="s")


@functools.partial(
    pl.kernel,
    mesh=_mesh,
    compiler_params=pltpu.CompilerParams(
        needs_layout_passes=False, use_tc_tiling_on_sc=False),
    out_type=jax.ShapeDtypeStruct((N_EDGES,), jnp.float32),
    scratch_types=[
        pltpu.VMEM((NCHUNK, CHUNK), jnp.int32),
        pltpu.VMEM((NCHUNK, CHUNK), jnp.int32),
        pltpu.VMEM((CHUNK, N_PAIR), jnp.float32),
        pltpu.VMEM((CHUNK, N_PAIR), jnp.float32),
        pltpu.VMEM((CHUNK, N_PAIR), jnp.float32),
        pltpu.VMEM((CHUNK, N_PAIR), jnp.float32),
        pltpu.VMEM((N_PAIR, 32), jnp.bfloat16),
        pltpu.VMEM((EDGES_PER_WORKER,), jnp.float32),
        pltpu.VMEM((STAGE_ROWS, N_PAIR), jnp.float32),
        pltpu.VMEM_SHARED((N_NODES, N_PAIR), jnp.float32),
        pltpu.SemaphoreType.DMA((2,)),
        pltpu.SemaphoreType.DMA((2,)),
    ],
)
def _decoder_sc(nf_hbm, r_hbm, c_hbm, wb_hbm, out_hbm,
                ri_v, ci_v, rr0, rr1, cr0, cr1, wb_v, out_v, stage_v,
                table_sh, sem_r, sem_c):
    sid = lax.axis_index("s")
    wid = sid * NUM_CORES + lax.axis_index("c")
    pltpu.sync_copy(wb_hbm, wb_v)
    pltpu.sync_copy(r_hbm.at[wid], ri_v)
    pltpu.sync_copy(c_hbm.at[wid], ci_v)

    def stage_body(k, carry):
        piece = sid + k * NUM_SUBCORES

        @pl.when(piece < N_STAGE_PIECES)
        def _():
            off = piece * STAGE_ROWS
            pltpu.sync_copy(nf_hbm.at[pl.ds(off, STAGE_ROWS)], stage_v)
            pltpu.sync_copy(stage_v, table_sh.at[pl.ds(off, STAGE_ROWS)])

        return carry

    lax.fori_loop(0, (N_STAGE_PIECES + NUM_SUBCORES - 1) // NUM_SUBCORES,
                  stage_body, 0)
    plsc.subcore_barrier()

    base = wid * EDGES_PER_WORKER
    lane = lax.iota(jnp.int32, 16)
    bufs = ((rr0, cr0), (rr1, cr1))

    def start_gathers(i, b):
        rr, cr = bufs[b]
        pltpu.async_copy(table_sh.at[ri_v.at[i]], rr, sem_r.at[b])
        pltpu.async_copy(table_sh.at[ci_v.at[i]], cr, sem_c.at[b])

    def wait_gathers(i, b):
        rr, cr = bufs[b]
        pltpu.make_async_copy(table_sh.at[ri_v.at[i]], rr,
                              sem_r.at[b]).wait()
        pltpu.make_async_copy(table_sh.at[ci_v.at[i]], cr,
                              sem_c.at[b]).wait()

    def compute_chunk(i, b):
        rr, cr = bufs[b]
        evecs = [lane + g * 16 for g in range(GROUPS)]

        def p_body(p, accs):
            wv = wb_v[p]
            pvec = (jnp.full((16,), p, dtype=jnp.int32) + lane) & (N_PAIR - 1)
            new = []
            for g in range(GROUPS):
                ar = plsc.load_gather(rr, [evecs[g], pvec])
                ac = plsc.load_gather(cr, [evecs[g], pvec])
                arb = plsc.bitcast(ar, jnp.bfloat16)
                acb = plsc.bitcast(ac, jnp.bfloat16)
                m = jnp.abs(arb - acb) * wv
                lo, hi = plsc.unpack(m, format=plsc.PackFormat.INTERLEAVED)
                new.append(accs[2 * g] + lo)
                new.append(accs[2 * g + 1] + hi)
            return tuple(new)

        zero = jnp.zeros((16,), jnp.float32)
        accs = lax.fori_loop(0, N_PAIR, p_body, (zero,) * (2 * GROUPS),
                             unroll=1)
        for g in range(GROUPS):
            plsc.store_scatter(out_v, [evecs[g] + i * CHUNK],
                               accs[2 * g] + accs[2 * g + 1])

    start_gathers(0, 0)

    def pair_body(j, carry):
        i0 = 2 * j
        start_gathers(i0 + 1, 1)
        wait_gathers(i0, 0)
        compute_chunk(i0, 0)
        start_gathers(i0 + 2, 0)
        wait_gathers(i0 + 1, 1)
        compute_chunk(i0 + 1, 1)
        return carry

    lax.fori_loop(0, (NCHUNK - 1) // 2, pair_body, 0)
    wait_gathers(NCHUNK - 1, 0)
    compute_chunk(NCHUNK - 1, 0)
    pltpu.sync_copy(out_v, out_hbm.at[pl.ds(base, EDGES_PER_WORKER)])


def kernel(node_features, r_indices, c_indices, w):
    r = r_indices.astype(jnp.int32).reshape(NUM_WORKERS, NCHUNK, CHUNK)
    c = c_indices.astype(jnp.int32).reshape(NUM_WORKERS, NCHUNK, CHUNK)
    nf_bf = node_features.astype(jnp.bfloat16)
    nf_packed = lax.bitcast_convert_type(
        nf_bf.reshape(N_NODES, N_PAIR, 2), jnp.float32)
    wf = w.reshape(D_FEAT).astype(jnp.bfloat16)
    q = (jnp.arange(N_PAIR)[:, None] + jnp.arange(16)[None, :]) % N_PAIR
    wb = jnp.stack([wf[2 * q], wf[2 * q + 1]], axis=-1).reshape(N_PAIR, 32)
    return _decoder_sc(nf_packed, r, c, wb)
